# baseline (device time: 241466 ns/iter reference)
import jax
import jax.numpy as jnp
from jax import lax
from jax.experimental import pallas as pl
from jax.experimental.pallas import tpu as pltpu

N_DEV = 4
KC = 1024
MH = 1024
NKC = 8
NH = 2
CPR = NKC * NH
NCHUNK = N_DEV * CPR
NSTEP = NCHUNK + 1

_SLOT = (0, 1, 2, 0)


def kernel(x, w_mat):
    M, K = x.shape
    _, N = w_mat.shape
    NB = N // N_DEV

    my = lax.axis_index("i")
    offs = jnp.array([1, 2, 3, 0], dtype=jnp.int32)
    targets = (my + offs) % N_DEV

    def body(targ_ref, x_ref, w_ref, dummy_ref, out_ref,
             acc_ref, x_bf, w_bf, send_bufs,
             send_sems, recv_sems, copy_sem):
        del dummy_ref
        s = pl.program_id(0)
        my_pos = lax.axis_index("i")

        @pl.when(s == 0)
        def _():
            barrier = pltpu.get_barrier_semaphore()
            for d in range(1, N_DEV):
                pl.semaphore_signal(
                    barrier, inc=1,
                    device_id=((my_pos + d) % N_DEV,),
                    device_id_type=pl.DeviceIdType.MESH,
                )
            pl.semaphore_wait(barrier, N_DEV - 1)

        x_bf[s % 2] = x_ref[...].astype(jnp.bfloat16)
        w_bf[s % 2] = w_ref[...].astype(jnp.bfloat16)

        c = (s - 1) % NSTEP
        kc = (c % CPR) // 2
        h = c % 2
        prod = jnp.dot(x_bf[(s - 1) % 2], w_bf[(s - 1) % 2],
                       preferred_element_type=jnp.float32)
        rows = pl.ds(h * MH, MH)
        prev = jnp.where(kc == 0, jnp.zeros_like(prod), acc_ref[rows, :])
        acc_ref[rows, :] = prod + prev

        def send_desc(r):
            return pltpu.make_async_remote_copy(
                src_ref=send_bufs.at[_SLOT[r]],
                dst_ref=out_ref.at[pl.ds(my_pos * M, M), :],
                send_sem=send_sems.at[_SLOT[r]],
                recv_sem=recv_sems.at[r],
                device_id=(targ_ref[r],),
                device_id_type=pl.DeviceIdType.MESH,
            )

        for tt in range(N_DEV):
            for hh in range(NH):
                @pl.when(s == tt * CPR + CPR - NH + hh + 1)
                def _(tt=tt, hh=hh):
                    slot = _SLOT[tt]
                    if hh == 0 and tt == N_DEV - 1:
                        send_desc(0).wait_send()
                    y = jax.nn.gelu(
                        acc_ref[pl.ds(hh * MH, MH), :], approximate=True)
                    send_bufs[slot, pl.ds(hh * MH, MH), :] = (
                        y.astype(jnp.bfloat16))

                    if hh == NH - 1:
                        if tt < N_DEV - 1:
                            send_desc(tt).start()
                        else:
                            own_copy = pltpu.make_async_copy(
                                send_bufs.at[slot],
                                out_ref.at[pl.ds(my_pos * M, M), :],
                                copy_sem,
                            )
                            own_copy.start()

                            for r in range(N_DEV - 1):
                                src = (my_pos - (r + 1)) % N_DEV
                                recv_desc = pltpu.make_async_remote_copy(
                                    src_ref=send_bufs.at[_SLOT[r]],
                                    dst_ref=out_ref.at[pl.ds(src * M, M), :],
                                    send_sem=send_sems.at[_SLOT[r]],
                                    recv_sem=recv_sems.at[r],
                                    device_id=(my_pos,),
                                    device_id_type=pl.DeviceIdType.MESH,
                                )
                                recv_desc.wait_recv()

                            send_desc(1).wait_send()
                            send_desc(2).wait_send()
                            own_copy.wait()

    def _chunk(sv):
        c = jnp.minimum(sv, NCHUNK - 1)
        return c // CPR, (c % CPR) // 2, c % 2

    def _x_map(sv, targ):
        _, kc, h = _chunk(sv)
        return h, kc

    def _w_map(sv, targ):
        t, kc, _ = _chunk(sv)
        return kc, targ[t]

    grid_spec = pltpu.PrefetchScalarGridSpec(
        num_scalar_prefetch=1,
        grid=(NSTEP,),
        in_specs=[
            pl.BlockSpec((MH, KC), _x_map),
            pl.BlockSpec((KC, NB), _w_map),
            pl.BlockSpec(memory_space=pl.ANY),
        ],
        out_specs=pl.BlockSpec(memory_space=pl.ANY),
        scratch_shapes=[
            pltpu.VMEM((M, NB), jnp.float32),
            pltpu.VMEM((2, MH, KC), jnp.bfloat16),
            pltpu.VMEM((2, KC, NB), jnp.bfloat16),
            pltpu.VMEM((3, M, NB), jnp.bfloat16),
            pltpu.SemaphoreType.DMA((3,)),
            pltpu.SemaphoreType.DMA((N_DEV - 1,)),
            pltpu.SemaphoreType.DMA,
        ],
    )

    dummy = pltpu.with_memory_space_constraint(
        jnp.zeros((N_DEV * M, NB), jnp.bfloat16), pltpu.MemorySpace.HBM)

    return pl.pallas_call(
        body,
        grid_spec=grid_spec,
        out_shape=jax.ShapeDtypeStruct((N_DEV * M, NB), jnp.bfloat16),
        input_output_aliases={3: 0},
        compiler_params=pltpu.CompilerParams(
            dimension_semantics=("arbitrary",),
            collective_id=0,
            vmem_limit_bytes=63 * 1024 * 1024,
        ),
    )(targets, x, w_mat, dummy)


# device time: 224034 ns/iter; 1.0778x vs baseline; 1.0778x over previous
import jax
import jax.numpy as jnp
from jax import lax
from jax.experimental import pallas as pl
from jax.experimental.pallas import tpu as pltpu

N_DEV = 4
KC = 1024


def kernel(x, w_mat):
    M, K = x.shape
    _, N = w_mat.shape
    NB = N // N_DEV
    NK = K // KC

    my = lax.axis_index("i")
    offs = jnp.array([1, 2, 3, 0], dtype=jnp.int32)
    targets = (my + offs) % N_DEV

    def body(targ_ref, x_ref, w_ref, dummy_ref, out_ref,
             acc_ref, send_bufs, send_sems, recv_sems, copy_sem):
        del dummy_ref
        t = pl.program_id(0)
        k = pl.program_id(1)
        my_pos = lax.axis_index("i")

        @pl.when((t == 0) & (k == 0))
        def _():
            barrier = pltpu.get_barrier_semaphore()
            for d in range(1, N_DEV):
                pl.semaphore_signal(
                    barrier, inc=1,
                    device_id=((my_pos + d) % N_DEV,),
                    device_id_type=pl.DeviceIdType.MESH,
                )
            pl.semaphore_wait(barrier, N_DEV - 1)

        prod = jnp.dot(x_ref[...].astype(jnp.bfloat16),
                       w_ref[...].astype(jnp.bfloat16),
                       preferred_element_type=jnp.float32)

        @pl.when(k == 0)
        def _():
            acc_ref[...] = prod

        @pl.when(k > 0)
        def _():
            acc_ref[...] += prod

        def send_desc(r):
            return pltpu.make_async_remote_copy(
                src_ref=send_bufs.at[r],
                dst_ref=out_ref.at[pl.ds(my_pos * M, M), :],
                send_sem=send_sems.at[r],
                recv_sem=recv_sems.at[r],
                device_id=(targ_ref[r],),
                device_id_type=pl.DeviceIdType.MESH,
            )

        for tt in range(N_DEV):
            @pl.when((k == NK - 1) & (t == tt))
            def _(tt=tt):
                y = jax.nn.gelu(acc_ref[...], approximate=True)

                if tt < N_DEV - 1:
                    send_bufs[tt] = y.astype(jnp.bfloat16)
                    send_desc(tt).start()
                else:
                    send_desc(0).wait_send()
                    send_bufs[0] = y.astype(jnp.bfloat16)
                    own_copy = pltpu.make_async_copy(
                        send_bufs.at[0],
                        out_ref.at[pl.ds(my_pos * M, M), :],
                        copy_sem,
                    )
                    own_copy.start()

                    for r in range(N_DEV - 1):
                        src = (my_pos - (r + 1)) % N_DEV
                        recv_desc = pltpu.make_async_remote_copy(
                            src_ref=send_bufs.at[r],
                            dst_ref=out_ref.at[pl.ds(src * M, M), :],
                            send_sem=send_sems.at[r],
                            recv_sem=recv_sems.at[r],
                            device_id=(my_pos,),
                            device_id_type=pl.DeviceIdType.MESH,
                        )
                        recv_desc.wait_recv()

                    for r in range(1, N_DEV - 1):
                        send_desc(r).wait_send()
                    own_copy.wait()

    grid_spec = pltpu.PrefetchScalarGridSpec(
        num_scalar_prefetch=1,
        grid=(N_DEV, NK),
        in_specs=[
            pl.BlockSpec((M, KC), lambda t, k, targ: (0, k)),
            pl.BlockSpec((KC, NB), lambda t, k, targ: (k, targ[t])),
            pl.BlockSpec(memory_space=pl.ANY),
        ],
        out_specs=pl.BlockSpec(memory_space=pl.ANY),
        scratch_shapes=[
            pltpu.VMEM((M, NB), jnp.float32),
            pltpu.VMEM((N_DEV - 1, M, NB), jnp.bfloat16),
            pltpu.SemaphoreType.DMA((N_DEV - 1,)),
            pltpu.SemaphoreType.DMA((N_DEV - 1,)),
            pltpu.SemaphoreType.DMA,
        ],
    )

    dummy = pltpu.with_memory_space_constraint(
        jnp.zeros((N_DEV * M, NB), jnp.bfloat16), pltpu.MemorySpace.HBM)

    return pl.pallas_call(
        body,
        grid_spec=grid_spec,
        out_shape=jax.ShapeDtypeStruct((N_DEV * M, NB), jnp.bfloat16),
        input_output_aliases={3: 0},
        compiler_params=pltpu.CompilerParams(
            dimension_semantics=("arbitrary", "arbitrary"),
            collective_id=0,
            vmem_limit_bytes=63 * 1024 * 1024,
        ),
    )(targets, x, w_mat, dummy)
